# baseline (speedup 1.0000x reference)
import functools

import jax
import jax.numpy as jnp
from jax import lax
from jax.experimental import pallas as pl
from jax.experimental.pallas import tpu as pltpu
from jax.experimental.pallas import tpu_sc as plsc

N = 10000
E = 320000
D = 128
NC = 2
NS = 16
NW = NC * NS
EPT = E // NW
C = 80
NCHUNK = EPT // C
RPS = N // NS
ZR = 125
BN = 2000


def _deg_body(src_h, dst_h, w_h, degp_h, srcf, dstf, wf, degt):
    cc = lax.axis_index("c")
    s = lax.axis_index("s")
    g = cc * NS + s
    gbase = pl.multiple_of(g * EPT, 16)
    pltpu.sync_copy(src_h.at[pl.ds(gbase, EPT)], srcf)
    pltpu.sync_copy(dst_h.at[pl.ds(gbase, EPT)], dstf)
    pltpu.sync_copy(w_h.at[pl.ds(gbase, EPT)], wf)
    zv = jnp.zeros((16,), jnp.float32)

    @pl.loop(0, N // 16)
    def _zero(i):
        degt[pl.ds(i * 16, 16)] = zv

    @pl.loop(0, EPT // 16)
    def _acc(i):
        sv = srcf[pl.ds(i * 16, 16)]
        dv = dstf[pl.ds(i * 16, 16)]
        wv = wf[pl.ds(i * 16, 16)]
        wp = jnp.where(sv == dv, 0.0, wv)
        plsc.addupdate_scatter(degt, [sv], wp)

    pltpu.sync_copy(degt, degp_h.at[g])


@functools.cache
def _deg_call():
    return pl.kernel(
        _deg_body,
        out_type=jax.ShapeDtypeStruct((NW, N), jnp.float32),
        mesh=plsc.VectorSubcoreMesh(core_axis_name="c", subcore_axis_name="s"),
        compiler_params=pltpu.CompilerParams(needs_layout_passes=False),
        scratch_types=[
            pltpu.VMEM((EPT,), jnp.int32),
            pltpu.VMEM((EPT,), jnp.int32),
            pltpu.VMEM((EPT,), jnp.float32),
            pltpu.VMEM((N,), jnp.float32),
        ],
    )


DR = 80
NB = N // DR
NSET = 3
NTRIP = (NCHUNK - 2) // NSET


def _agg_pass(zero_first, u_h, epk_h, out_h, wpb, dx0, dx1, dx2,
              eb0, eb1, eb2, rows0, rows1, rows2,
              zbuf, acc, gs0, gs1, gs2, ss0, ss1, ss2, es0, es1, es2):
    cc = lax.axis_index("c")
    s = lax.axis_index("s")
    g = cc * NS + s
    ebs = (eb0, eb1, eb2)
    dxs = (dx0, dx1, dx2)
    rowss = (rows0, rows1, rows2)
    gss = (gs0, gs1, gs2)
    sss = (ss0, ss1, ss2)
    ess = (es0, es1, es2)

    if zero_first:
        @pl.loop(s, NB, step=NS)
        def _zero(b):
            pltpu.sync_copy(zbuf, acc.at[pl.ds(pl.multiple_of(b * DR, 16), DR)])

        plsc.subcore_barrier()

    def scatter_desc(t):
        return pltpu.make_async_copy(rowss[t], acc.at[dxs[t]], sss[t])

    def edge_desc(j, t):
        return pltpu.make_async_copy(epk_h.at[g * NCHUNK + j], ebs[t], ess[t])

    def gather_desc(t):
        return pltpu.make_async_copy(u_h.at[ebs[t].at[0]], rowss[t], gss[t])

    def step(j, t, jj=None, last=0):
        t2 = (t + 2) % NSET
        if last == 0:
            pltpu.async_copy(epk_h.at[g * NCHUNK + j + 2], ebs[t2], ess[t2])
        gather_desc(t).wait()

        rows = rowss[t]
        eb = ebs[t]
        dx = dxs[t]
        for k in range(C // 16):
            sv = eb[0, pl.ds(k * 16, 16)]
            dv = eb[1, pl.ds(k * 16, 16)]
            wv = plsc.bitcast(eb[2, pl.ds(k * 16, 16)], jnp.float32)
            dx[pl.ds(k * 16, 16)] = dv
            wpb[pl.ds(k * 16, 16)] = jnp.where(sv == dv, 0.0, wv)

        @pl.loop(0, C // 16)
        def _scale(tt):
            wv = wpb[pl.ds(tt * 16, 16)]
            for e16 in range(16):
                ws = wv[e16]
                e = tt * 16 + e16
                for q in range(D // 16):
                    rows[e, pl.ds(q * 16, 16)] = rows[e, pl.ds(q * 16, 16)] * ws

        pltpu.async_copy(rows, acc.at[dx], sss[t], add=True)

        if last == 0:
            edge_desc(j + 2, t2).wait()
            if jj is None:
                scatter_desc(t2).wait()
            else:
                @pl.when(jj > 0)
                def _w():
                    scatter_desc(t2).wait()
            pltpu.async_copy(u_h.at[ebs[t2].at[0]], rowss[t2], gss[t2])

    pltpu.sync_copy(epk_h.at[g * NCHUNK], ebs[0])
    pltpu.async_copy(epk_h.at[g * NCHUNK + 1], ebs[1], ess[1])
    pltpu.async_copy(u_h.at[ebs[0].at[0]], rows0, gss[0])
    edge_desc(1, 1).wait()
    pltpu.async_copy(u_h.at[ebs[1].at[0]], rows1, gss[1])

    @pl.loop(0, NTRIP)
    def _chunks(jj):
        base = jj * NSET
        step(base, 0, jj=jj)
        step(base + 1, 1)
        step(base + 2, 2)

    step(NCHUNK - 2, 0, last=1)
    step(NCHUNK - 1, 1, last=2)
    scatter_desc(2).wait()
    scatter_desc(0).wait()
    scatter_desc(1).wait()

    plsc.subcore_barrier()

    @pl.loop(s, NB, step=NS)
    def _drain(b):
        off = pl.multiple_of(b * DR, 16)
        pltpu.sync_copy(acc.at[pl.ds(off, DR)], out_h.at[cc, pl.ds(off, DR)])
        pltpu.sync_copy(zbuf, acc.at[pl.ds(off, DR)])

    plsc.subcore_barrier()


@functools.cache
def _agg_call(P):
    def body(*refs):
        us = refs[:P]
        epk_h = refs[P]
        outs = refs[P + 1:P + 1 + P]
        scratch = refs[P + 1 + P:]
        zbuf = scratch[10]
        zv = jnp.zeros((16,), jnp.float32)

        @pl.loop(0, DR)
        def _zz(r):
            for q in range(D // 16):
                zbuf[r, pl.ds(q * 16, 16)] = zv

        for p in range(P):
            _agg_pass(p == 0, us[p], epk_h, outs[p], *scratch)

    return pl.kernel(
        body,
        out_type=tuple(jax.ShapeDtypeStruct((NC, N, D), jnp.float32)
                       for _ in range(P)),
        mesh=plsc.VectorSubcoreMesh(core_axis_name="c", subcore_axis_name="s"),
        compiler_params=pltpu.CompilerParams(needs_layout_passes=False),
        scratch_types=[
            pltpu.VMEM((C,), jnp.float32),
            pltpu.VMEM((C,), jnp.int32),
            pltpu.VMEM((C,), jnp.int32),
            pltpu.VMEM((C,), jnp.int32),
            pltpu.VMEM((3, C), jnp.int32),
            pltpu.VMEM((3, C), jnp.int32),
            pltpu.VMEM((3, C), jnp.int32),
            pltpu.VMEM((C, D), jnp.float32),
            pltpu.VMEM((C, D), jnp.float32),
            pltpu.VMEM((C, D), jnp.float32),
            pltpu.VMEM((DR, D), jnp.float32),
            pltpu.VMEM_SHARED((N, D), jnp.float32),
            pltpu.SemaphoreType.DMA,
            pltpu.SemaphoreType.DMA,
            pltpu.SemaphoreType.DMA,
            pltpu.SemaphoreType.DMA,
            pltpu.SemaphoreType.DMA,
            pltpu.SemaphoreType.DMA,
            pltpu.SemaphoreType.DMA,
            pltpu.SemaphoreType.DMA,
            pltpu.SemaphoreType.DMA,
        ],
    )


def _stage_a1(dpt_ref, x_ref, h_ref, sc_ref, ux_ref, uh_ref, dvec_ref):
    deg = jnp.sum(dpt_ref[...], axis=1)
    dinv = jnp.where(deg > 0, lax.rsqrt(jnp.maximum(deg, 1e-12)), 0.0)
    c = sc_ref[0, 0]
    di = dinv[:, None]
    ux_ref[...] = di * x_ref[...]
    uh_ref[...] = di * h_ref[...]
    dvec_ref[...] = jnp.stack([dinv, -c * dinv], axis=1)


def _stage_a2(x_ref, h_ref, w_ref, p_ref):
    w = w_ref[...]
    bf = jnp.bfloat16
    p_ref[...] = (
        jnp.dot(x_ref[...].astype(bf), w[:D], preferred_element_type=jnp.float32)
        + jnp.dot(h_ref[...].astype(bf), w[D:], preferred_element_type=jnp.float32))


def _stage_b(h_ref, p_ref, sx_ref, sh_ref, dvec_ref, wagg_ref, w2_ref, b3_ref,
             z_ref, u2_ref, ph_ref):
    sd = dvec_ref[...][:, 1:2]
    p = p_ref[...]
    b3 = b3_ref[...]
    wagg = wagg_ref[...]
    bf = jnp.bfloat16
    asx = (sd * (sx_ref[0] + sx_ref[1])).astype(bf)
    ash = (sd * (sh_ref[0] + sh_ref[1])).astype(bf)
    m1 = (jnp.dot(asx, wagg[:D], preferred_element_type=jnp.float32)
          + jnp.dot(ash, wagg[D:], preferred_element_type=jnp.float32))
    zg = jax.nn.sigmoid(p[:, 0:D] + m1[:, 0:D] + b3[0:1, :])
    rg = jax.nn.sigmoid(p[:, D:2 * D] + m1[:, D:2 * D] + b3[1:2, :])
    hr = h_ref[...] * rg
    m = jnp.dot(hr.astype(bf), w2_ref[...], preferred_element_type=jnp.float32)
    z_ref[...] = zg
    u2_ref[...] = sd * m[:, 0:D]
    ph_ref[...] = p[:, 2 * D:3 * D] + m1[:, 2 * D:3 * D] \
        + m[:, D:2 * D] + b3[2:3, :]


def _stage_c(a2_ref, ph_ref, z_ref, h_ref, dvec_ref, wl_ref, bl_ref,
             hout_ref, y_ref):
    di = dvec_ref[...][:, 0:1]
    ht = jnp.tanh(ph_ref[...] + di * (a2_ref[0] + a2_ref[1]))
    zg = z_ref[...]
    hh = zg * h_ref[...] + (1.0 - zg) * ht
    hout_ref[...] = hh
    y_ref[...] = (jnp.dot(jnp.maximum(hh, 0.0), wl_ref[...],
                          preferred_element_type=jnp.float32) + bl_ref[...])


def _row_spec(shape):
    nb = len(shape)
    if nb == 2:
        return pl.BlockSpec((BN, shape[1]), lambda i: (i, 0))
    return pl.BlockSpec((shape[0], BN, shape[2]), lambda i: (0, i, 0))


def _full_spec(shape):
    nd = len(shape)
    return pl.BlockSpec(shape, lambda i: (0,) * nd)


_GRID = (N // BN,)


def _call_a1(degpt, x, h0, scal):
    return pl.pallas_call(
        _stage_a1,
        grid=_GRID,
        in_specs=[
            _row_spec((N, NW)), _row_spec((N, D)), _row_spec((N, D)),
            _full_spec((1, 1)),
        ],
        out_specs=[_row_spec((N, D)), _row_spec((N, D)), _row_spec((N, 2))],
        out_shape=[jax.ShapeDtypeStruct((N, D), jnp.float32),
                   jax.ShapeDtypeStruct((N, D), jnp.float32),
                   jax.ShapeDtypeStruct((N, 2), jnp.float32)],
    )(degpt, x, h0, scal)


def _call_a2(x, h0, wbig):
    return pl.pallas_call(
        _stage_a2,
        grid=_GRID,
        in_specs=[
            _row_spec((N, D)), _row_spec((N, D)),
            _full_spec((2 * D, 3 * D)),
        ],
        out_specs=[_row_spec((N, 3 * D))],
        out_shape=[jax.ShapeDtypeStruct((N, 3 * D), jnp.float32)],
    )(x, h0, wbig)[0]


def _call_b(h0, p, sx, sh, dvec, wagg, w2, b3):
    return pl.pallas_call(
        _stage_b,
        grid=_GRID,
        in_specs=[
            _row_spec((N, D)), _row_spec((N, 3 * D)),
            _row_spec((NC, N, D)), _row_spec((NC, N, D)),
            _row_spec((N, 2)), _full_spec((2 * D, 3 * D)),
            _full_spec((D, 2 * D)), _full_spec((3, D)),
        ],
        out_specs=[_row_spec((N, D)), _row_spec((N, D)), _row_spec((N, D))],
        out_shape=[jax.ShapeDtypeStruct((N, D), jnp.float32),
                   jax.ShapeDtypeStruct((N, D), jnp.float32),
                   jax.ShapeDtypeStruct((N, D), jnp.float32)],
    )(h0, p, sx, sh, dvec, wagg, w2, b3)


def _call_c(a2, ph, z, h0, dvec, wl, bl):
    return pl.pallas_call(
        _stage_c,
        grid=_GRID,
        in_specs=[
            _row_spec((NC, N, D)), _row_spec((N, D)), _row_spec((N, D)),
            _row_spec((N, D)), _row_spec((N, 2)),
            _full_spec((D, D)), _full_spec((1, D)),
        ],
        out_specs=[_row_spec((N, D)), _row_spec((N, D))],
        out_shape=[jax.ShapeDtypeStruct((N, D), jnp.float32),
                   jax.ShapeDtypeStruct((N, D), jnp.float32)],
    )(a2, ph, z, h0, dvec, wl, bl)


def kernel(x, edge_index, edge_weight, h0,
           W_xz, b_xz, W_hz, b_hz, W_xr, b_xr, W_hr, b_hr,
           W_xh, b_xh, W_hh, b_hh, W_lin, b_lin, lambda_max=2.0):
    f32 = jnp.float32
    lam = jnp.asarray(lambda_max, f32)
    c = 2.0 / lam
    diag = c - 1.0

    src1 = edge_index[0].reshape(E)
    dst1 = edge_index[1].reshape(E)
    w1 = edge_weight.astype(f32).reshape(E)
    epk = jnp.stack([
        src1.reshape(NW * NCHUNK, C),
        dst1.reshape(NW * NCHUNK, C),
        lax.bitcast_convert_type(w1, jnp.int32).reshape(NW * NCHUNK, C),
    ], axis=1)

    degp = _deg_call()(src1, dst1, w1)

    zb = jnp.zeros((D, D), f32)
    x_rows = jnp.concatenate([
        W_xz[0] + diag * W_xz[1],
        W_xr[0] + diag * W_xr[1],
        W_xh[0] + diag * W_xh[1],
    ], axis=1)
    h_rows = jnp.concatenate([
        W_hz[0] + diag * W_hz[1],
        W_hr[0] + diag * W_hr[1],
        zb,
    ], axis=1)
    wbig = jnp.concatenate([x_rows, h_rows], axis=0).astype(jnp.bfloat16)
    wagg = jnp.concatenate([
        jnp.concatenate([W_xz[1], W_xr[1], W_xh[1]], axis=1),
        jnp.concatenate([W_hz[1], W_hr[1], zb], axis=1),
    ], axis=0).astype(jnp.bfloat16)
    scal = jnp.reshape(c, (1, 1))

    ux, uh, dvec = _call_a1(degp.T, x, h0, scal)

    sx, sh = _agg_call(2)(ux, uh, epk)
    p_all = _call_a2(x, h0, wbig)

    w2b = jnp.concatenate([W_hh[1], W_hh[0] + diag * W_hh[1]],
                          axis=1).astype(jnp.bfloat16)
    b3 = jnp.stack([b_xz + b_hz, b_xr + b_hr, b_xh + b_hh], axis=0)

    z, u2, ph = _call_b(h0, p_all, sx, sh, dvec, wagg, w2b, b3)

    (a2,) = _agg_call(1)(u2, epk)

    wl = jnp.zeros((D, D), f32).at[:, :W_lin.shape[1]].set(W_lin)
    bl = jnp.zeros((1, D), f32).at[0, :b_lin.shape[0]].set(b_lin)

    h_new, y_pad = _call_c(a2, ph, z, h0, dvec, wl, bl)

    return (lax.stop_gradient(h_new), y_pad[:, :W_lin.shape[1]])


# speedup vs baseline: 1.0051x; 1.0051x over previous
import functools

import jax
import jax.numpy as jnp
from jax import lax
from jax.experimental import pallas as pl
from jax.experimental.pallas import tpu as pltpu
from jax.experimental.pallas import tpu_sc as plsc

N = 10000
E = 320000
D = 128
NC = 2
NS = 16
NW = NC * NS
EPT = E // NW
C = 80
NCHUNK = EPT // C
RPS = N // NS
ZR = 125
BN = 2000


def _deg_body(src_h, dst_h, w_h, degp_h, srcf, dstf, wf, degt):
    cc = lax.axis_index("c")
    s = lax.axis_index("s")
    g = cc * NS + s
    gbase = pl.multiple_of(g * EPT, 16)
    pltpu.sync_copy(src_h.at[pl.ds(gbase, EPT)], srcf)
    pltpu.sync_copy(dst_h.at[pl.ds(gbase, EPT)], dstf)
    pltpu.sync_copy(w_h.at[pl.ds(gbase, EPT)], wf)
    zv = jnp.zeros((16,), jnp.float32)

    @pl.loop(0, N // 16)
    def _zero(i):
        degt[pl.ds(i * 16, 16)] = zv

    @pl.loop(0, EPT // 16)
    def _acc(i):
        sv = srcf[pl.ds(i * 16, 16)]
        dv = dstf[pl.ds(i * 16, 16)]
        wv = wf[pl.ds(i * 16, 16)]
        wp = jnp.where(sv == dv, 0.0, wv)
        plsc.addupdate_scatter(degt, [sv], wp)

    pltpu.sync_copy(degt, degp_h.at[g])


@functools.cache
def _deg_call():
    return pl.kernel(
        _deg_body,
        out_type=jax.ShapeDtypeStruct((NW, N), jnp.float32),
        mesh=plsc.VectorSubcoreMesh(core_axis_name="c", subcore_axis_name="s"),
        compiler_params=pltpu.CompilerParams(needs_layout_passes=False),
        scratch_types=[
            pltpu.VMEM((EPT,), jnp.int32),
            pltpu.VMEM((EPT,), jnp.int32),
            pltpu.VMEM((EPT,), jnp.float32),
            pltpu.VMEM((N,), jnp.float32),
        ],
    )


DR = 80
NB = N // DR
NSET = 3
NTRIP = (NCHUNK - 2) // NSET


def _agg_pass(u_h, epk_h, out_h, wpb, dx0, dx1, dx2,
              eb0, eb1, eb2, rows0, rows1, rows2,
              zbuf, acc, gs0, gs1, gs2, ss0, ss1, ss2, es0, es1, es2):
    cc = lax.axis_index("c")
    s = lax.axis_index("s")
    g = cc * NS + s
    ebs = (eb0, eb1, eb2)
    dxs = (dx0, dx1, dx2)
    rowss = (rows0, rows1, rows2)
    gss = (gs0, gs1, gs2)
    sss = (ss0, ss1, ss2)
    ess = (es0, es1, es2)

    zv = jnp.zeros((16,), jnp.float32)

    @pl.loop(0, DR)
    def _zz(r):
        for q in range(D // 16):
            zbuf[r, pl.ds(q * 16, 16)] = zv

    @pl.loop(s, NB, step=NS)
    def _zero(b):
        pltpu.sync_copy(zbuf, acc.at[pl.ds(pl.multiple_of(b * DR, 16), DR)])

    plsc.subcore_barrier()

    def scatter_desc(t):
        return pltpu.make_async_copy(rowss[t], acc.at[dxs[t]], sss[t])

    def edge_desc(j, t):
        return pltpu.make_async_copy(epk_h.at[g * NCHUNK + j], ebs[t], ess[t])

    def gather_desc(t):
        return pltpu.make_async_copy(u_h.at[ebs[t].at[0]], rowss[t], gss[t])

    def step(j, t, jj=None, last=0):
        t2 = (t + 2) % NSET
        if last == 0:
            pltpu.async_copy(epk_h.at[g * NCHUNK + j + 2], ebs[t2], ess[t2])
        gather_desc(t).wait()

        rows = rowss[t]
        eb = ebs[t]
        dx = dxs[t]
        for k in range(C // 16):
            sv = eb[0, pl.ds(k * 16, 16)]
            dv = eb[1, pl.ds(k * 16, 16)]
            wv = plsc.bitcast(eb[2, pl.ds(k * 16, 16)], jnp.float32)
            dx[pl.ds(k * 16, 16)] = dv
            wpb[pl.ds(k * 16, 16)] = jnp.where(sv == dv, 0.0, wv)

        @pl.loop(0, C // 16)
        def _scale(tt):
            wv = wpb[pl.ds(tt * 16, 16)]
            for e16 in range(16):
                ws = wv[e16]
                e = tt * 16 + e16
                for q in range(D // 16):
                    rows[e, pl.ds(q * 16, 16)] = rows[e, pl.ds(q * 16, 16)] * ws

        pltpu.async_copy(rows, acc.at[dx], sss[t], add=True)

        if last == 0:
            edge_desc(j + 2, t2).wait()
            if jj is None:
                scatter_desc(t2).wait()
            else:
                @pl.when(jj > 0)
                def _w():
                    scatter_desc(t2).wait()
            pltpu.async_copy(u_h.at[ebs[t2].at[0]], rowss[t2], gss[t2])

    pltpu.sync_copy(epk_h.at[g * NCHUNK], ebs[0])
    pltpu.async_copy(epk_h.at[g * NCHUNK + 1], ebs[1], ess[1])
    pltpu.async_copy(u_h.at[ebs[0].at[0]], rows0, gss[0])
    edge_desc(1, 1).wait()
    pltpu.async_copy(u_h.at[ebs[1].at[0]], rows1, gss[1])

    @pl.loop(0, NTRIP)
    def _chunks(jj):
        base = jj * NSET
        step(base, 0, jj=jj)
        step(base + 1, 1)
        step(base + 2, 2)

    step(NCHUNK - 2, 0, last=1)
    step(NCHUNK - 1, 1, last=2)
    scatter_desc(2).wait()
    scatter_desc(0).wait()
    scatter_desc(1).wait()

    plsc.subcore_barrier()

    @pl.loop(s, NB, step=NS)
    def _drain(b):
        off = pl.multiple_of(b * DR, 16)
        pltpu.sync_copy(acc.at[pl.ds(off, DR)], zbuf)
        pltpu.sync_copy(zbuf, out_h.at[cc, pl.ds(off, DR)])

    plsc.subcore_barrier()


@functools.cache
def _agg_call(P):
    def body(*refs):
        us = refs[:P]
        epk_h = refs[P]
        outs = refs[P + 1:P + 1 + P]
        scratch = refs[P + 1 + P:]
        for p in range(P):
            _agg_pass(us[p], epk_h, outs[p], *scratch)

    return pl.kernel(
        body,
        out_type=tuple(jax.ShapeDtypeStruct((NC, N, D), jnp.float32)
                       for _ in range(P)),
        mesh=plsc.VectorSubcoreMesh(core_axis_name="c", subcore_axis_name="s"),
        compiler_params=pltpu.CompilerParams(needs_layout_passes=False),
        scratch_types=[
            pltpu.VMEM((C,), jnp.float32),
            pltpu.VMEM((C,), jnp.int32),
            pltpu.VMEM((C,), jnp.int32),
            pltpu.VMEM((C,), jnp.int32),
            pltpu.VMEM((3, C), jnp.int32),
            pltpu.VMEM((3, C), jnp.int32),
            pltpu.VMEM((3, C), jnp.int32),
            pltpu.VMEM((C, D), jnp.float32),
            pltpu.VMEM((C, D), jnp.float32),
            pltpu.VMEM((C, D), jnp.float32),
            pltpu.VMEM((DR, D), jnp.float32),
            pltpu.VMEM_SHARED((N, D), jnp.float32),
            pltpu.SemaphoreType.DMA,
            pltpu.SemaphoreType.DMA,
            pltpu.SemaphoreType.DMA,
            pltpu.SemaphoreType.DMA,
            pltpu.SemaphoreType.DMA,
            pltpu.SemaphoreType.DMA,
            pltpu.SemaphoreType.DMA,
            pltpu.SemaphoreType.DMA,
            pltpu.SemaphoreType.DMA,
        ],
    )


def _stage_a1(dpt_ref, x_ref, h_ref, sc_ref, ux_ref, uh_ref, dvec_ref):
    deg = jnp.sum(dpt_ref[...], axis=1)
    dinv = jnp.where(deg > 0, lax.rsqrt(jnp.maximum(deg, 1e-12)), 0.0)
    c = sc_ref[0, 0]
    di = dinv[:, None]
    ux_ref[...] = di * x_ref[...]
    uh_ref[...] = di * h_ref[...]
    dvec_ref[...] = jnp.stack([dinv, -c * dinv], axis=1)


def _stage_a2(x_ref, h_ref, w_ref, p_ref):
    w = w_ref[...]
    bf = jnp.bfloat16
    p_ref[...] = (
        jnp.dot(x_ref[...].astype(bf), w[:D], preferred_element_type=jnp.float32)
        + jnp.dot(h_ref[...].astype(bf), w[D:], preferred_element_type=jnp.float32))


def _stage_b(h_ref, p_ref, sx_ref, sh_ref, dvec_ref, wagg_ref, w2_ref, b3_ref,
             z_ref, u2_ref, ph_ref):
    sd = dvec_ref[...][:, 1:2]
    p = p_ref[...]
    b3 = b3_ref[...]
    wagg = wagg_ref[...]
    bf = jnp.bfloat16
    asx = (sd * (sx_ref[0] + sx_ref[1])).astype(bf)
    ash = (sd * (sh_ref[0] + sh_ref[1])).astype(bf)
    m1 = (jnp.dot(asx, wagg[:D], preferred_element_type=jnp.float32)
          + jnp.dot(ash, wagg[D:], preferred_element_type=jnp.float32))
    zg = jax.nn.sigmoid(p[:, 0:D] + m1[:, 0:D] + b3[0:1, :])
    rg = jax.nn.sigmoid(p[:, D:2 * D] + m1[:, D:2 * D] + b3[1:2, :])
    hr = h_ref[...] * rg
    m = jnp.dot(hr.astype(bf), w2_ref[...], preferred_element_type=jnp.float32)
    z_ref[...] = zg
    u2_ref[...] = sd * m[:, 0:D]
    ph_ref[...] = p[:, 2 * D:3 * D] + m1[:, 2 * D:3 * D] \
        + m[:, D:2 * D] + b3[2:3, :]


def _stage_c(a2_ref, ph_ref, z_ref, h_ref, dvec_ref, wl_ref, bl_ref,
             hout_ref, y_ref):
    di = dvec_ref[...][:, 0:1]
    ht = jnp.tanh(ph_ref[...] + di * (a2_ref[0] + a2_ref[1]))
    zg = z_ref[...]
    hh = zg * h_ref[...] + (1.0 - zg) * ht
    hout_ref[...] = hh
    y_ref[...] = (jnp.dot(jnp.maximum(hh, 0.0), wl_ref[...],
                          preferred_element_type=jnp.float32) + bl_ref[...])


def _row_spec(shape):
    nb = len(shape)
    if nb == 2:
        return pl.BlockSpec((BN, shape[1]), lambda i: (i, 0))
    return pl.BlockSpec((shape[0], BN, shape[2]), lambda i: (0, i, 0))


def _full_spec(shape):
    nd = len(shape)
    return pl.BlockSpec(shape, lambda i: (0,) * nd)


_GRID = (N // BN,)


def _call_a1(degpt, x, h0, scal):
    return pl.pallas_call(
        _stage_a1,
        grid=_GRID,
        in_specs=[
            _row_spec((N, NW)), _row_spec((N, D)), _row_spec((N, D)),
            _full_spec((1, 1)),
        ],
        out_specs=[_row_spec((N, D)), _row_spec((N, D)), _row_spec((N, 2))],
        out_shape=[jax.ShapeDtypeStruct((N, D), jnp.float32),
                   jax.ShapeDtypeStruct((N, D), jnp.float32),
                   jax.ShapeDtypeStruct((N, 2), jnp.float32)],
    )(degpt, x, h0, scal)


def _call_a2(x, h0, wbig):
    return pl.pallas_call(
        _stage_a2,
        grid=_GRID,
        in_specs=[
            _row_spec((N, D)), _row_spec((N, D)),
            _full_spec((2 * D, 3 * D)),
        ],
        out_specs=[_row_spec((N, 3 * D))],
        out_shape=[jax.ShapeDtypeStruct((N, 3 * D), jnp.float32)],
    )(x, h0, wbig)[0]


def _call_b(h0, p, sx, sh, dvec, wagg, w2, b3):
    return pl.pallas_call(
        _stage_b,
        grid=_GRID,
        in_specs=[
            _row_spec((N, D)), _row_spec((N, 3 * D)),
            _row_spec((NC, N, D)), _row_spec((NC, N, D)),
            _row_spec((N, 2)), _full_spec((2 * D, 3 * D)),
            _full_spec((D, 2 * D)), _full_spec((3, D)),
        ],
        out_specs=[_row_spec((N, D)), _row_spec((N, D)), _row_spec((N, D))],
        out_shape=[jax.ShapeDtypeStruct((N, D), jnp.float32),
                   jax.ShapeDtypeStruct((N, D), jnp.float32),
                   jax.ShapeDtypeStruct((N, D), jnp.float32)],
    )(h0, p, sx, sh, dvec, wagg, w2, b3)


def _call_c(a2, ph, z, h0, dvec, wl, bl):
    return pl.pallas_call(
        _stage_c,
        grid=_GRID,
        in_specs=[
            _row_spec((NC, N, D)), _row_spec((N, D)), _row_spec((N, D)),
            _row_spec((N, D)), _row_spec((N, 2)),
            _full_spec((D, D)), _full_spec((1, D)),
        ],
        out_specs=[_row_spec((N, D)), _row_spec((N, D))],
        out_shape=[jax.ShapeDtypeStruct((N, D), jnp.float32),
                   jax.ShapeDtypeStruct((N, D), jnp.float32)],
    )(a2, ph, z, h0, dvec, wl, bl)


def kernel(x, edge_index, edge_weight, h0,
           W_xz, b_xz, W_hz, b_hz, W_xr, b_xr, W_hr, b_hr,
           W_xh, b_xh, W_hh, b_hh, W_lin, b_lin, lambda_max=2.0):
    f32 = jnp.float32
    lam = jnp.asarray(lambda_max, f32)
    c = 2.0 / lam
    diag = c - 1.0

    src1 = edge_index[0].reshape(E)
    dst1 = edge_index[1].reshape(E)
    w1 = edge_weight.astype(f32).reshape(E)
    epk = jnp.stack([
        src1.reshape(NW * NCHUNK, C),
        dst1.reshape(NW * NCHUNK, C),
        lax.bitcast_convert_type(w1, jnp.int32).reshape(NW * NCHUNK, C),
    ], axis=1)

    degp = _deg_call()(src1, dst1, w1)

    zb = jnp.zeros((D, D), f32)
    x_rows = jnp.concatenate([
        W_xz[0] + diag * W_xz[1],
        W_xr[0] + diag * W_xr[1],
        W_xh[0] + diag * W_xh[1],
    ], axis=1)
    h_rows = jnp.concatenate([
        W_hz[0] + diag * W_hz[1],
        W_hr[0] + diag * W_hr[1],
        zb,
    ], axis=1)
    wbig = jnp.concatenate([x_rows, h_rows], axis=0).astype(jnp.bfloat16)
    wagg = jnp.concatenate([
        jnp.concatenate([W_xz[1], W_xr[1], W_xh[1]], axis=1),
        jnp.concatenate([W_hz[1], W_hr[1], zb], axis=1),
    ], axis=0).astype(jnp.bfloat16)
    scal = jnp.reshape(c, (1, 1))

    ux, uh, dvec = _call_a1(degp.T, x, h0, scal)

    sx, sh = _agg_call(2)(ux, uh, epk)
    p_all = _call_a2(x, h0, wbig)

    w2b = jnp.concatenate([W_hh[1], W_hh[0] + diag * W_hh[1]],
                          axis=1).astype(jnp.bfloat16)
    b3 = jnp.stack([b_xz + b_hz, b_xr + b_hr, b_xh + b_hh], axis=0)

    z, u2, ph = _call_b(h0, p_all, sx, sh, dvec, wagg, w2b, b3)

    (a2,) = _agg_call(1)(u2, epk)

    wl = jnp.zeros((D, D), f32).at[:, :W_lin.shape[1]].set(W_lin)
    bl = jnp.zeros((1, D), f32).at[0, :b_lin.shape[0]].set(b_lin)

    h_new, y_pad = _call_c(a2, ph, z, h0, dvec, wl, bl)

    return (lax.stop_gradient(h_new), y_pad[:, :W_lin.shape[1]])
